# baseline (speedup 1.0000x reference)
import jax
import jax.numpy as jnp
from jax.experimental import pallas as pl

NODE_TYPES = ['flow', 'time', 'attr']
ETS = [('flow', 'acts_in', 'time', 2), ('time', 'evolves_to', 'time', 2), ('flow', 'interacts', 'flow', 5), ('flow', 'uses', 'attr', 1)]
H = 4
C = 128


def _layer_norm(x, g, b):
    mu = jnp.mean(x, axis=-1, keepdims=True)
    var = jnp.var(x, axis=-1, keepdims=True)
    return (x - mu) / jnp.sqrt(var + 1e-5) * g + b


def _sage(x_src, x_dst, ei, p):
    src, dst = ei[0], ei[1]
    n_dst = x_dst.shape[0]
    msg = jnp.take(x_src, src, axis=0)
    s = jax.ops.segment_sum(msg, dst, num_segments=n_dst)
    cnt = jax.ops.segment_sum(jnp.ones((src.shape[0],), jnp.float32), dst, num_segments=n_dst)
    mean = s / jnp.clip(cnt, 1.0, None)[:, None]
    return mean @ p['Wl'] + p['bl'] + x_dst @ p['Wr']


def _gatv2(x_src, x_dst, ei, ea, p):
    src, dst = ei[0], ei[1]
    n_dst = x_dst.shape[0]
    xl = (x_src @ p['Wl']).reshape(-1, H, C)
    xr = (x_dst @ p['Wr']).reshape(-1, H, C)
    z = jnp.take(xl, src, axis=0) + jnp.take(xr, dst, axis=0)
    z = z + (ea @ p['We']).reshape(-1, H, C)
    z = jax.nn.leaky_relu(z, 0.2)
    alpha = jnp.sum(z * p['att'][None, :, :], axis=-1)
    m = jax.ops.segment_max(alpha, dst, num_segments=n_dst)
    m = jnp.where(jnp.isfinite(m), m, 0.0)
    a = jnp.exp(alpha - jnp.take(m, dst, axis=0))
    ssum = jax.ops.segment_sum(a, dst, num_segments=n_dst)
    a = a / (jnp.take(ssum, dst, axis=0) + 1e-16)
    out = jax.ops.segment_sum(a[:, :, None] * jnp.take(xl, src, axis=0), dst, num_segments=n_dst)
    return jnp.mean(out, axis=1) + p['bias']


def kernel(x_flow, x_time, x_attr, ei_acts_in, ei_evolves_to, ei_interacts, ei_uses, ea_acts_in, ea_evolves_to, ea_interacts, ea_uses, params):
    eis = {'acts_in': ei_acts_in, 'evolves_to': ei_evolves_to, 'interacts': ei_interacts, 'uses': ei_uses}
    eas = {'acts_in': ea_acts_in, 'evolves_to': ea_evolves_to, 'interacts': ea_interacts, 'uses': ea_uses}
    x = {'flow': x_flow @ params['in']['flow']['W'] + params['in']['flow']['b'],
         'time': x_time @ params['in']['time']['W'] + params['in']['time']['b'],
         'attr': x_attr @ params['in']['attr']['W'] + params['in']['attr']['b']}
    layer_idx = 0
    for li in range(2):
        out = {}
        for (s, name, d, ed) in ETS:
            o = _sage(x[s], x[d], eis[name], params['sage'][li][name])
            out[d] = out.get(d, 0.0) + o
        x = {k: jax.nn.gelu(_layer_norm(x[k] + v, params['norm'][layer_idx][k]['g'], params['norm'][layer_idx][k]['b']), approximate=False) for k, v in out.items()}
        layer_idx += 1
    for li in range(2):
        out = {}
        for (s, name, d, ed) in ETS:
            o = _gatv2(x[s], x[d], eis[name], eas[name], params['gat'][li][name])
            out[d] = out.get(d, 0.0) + o
        x = {k: jax.nn.gelu(_layer_norm(x[k] + v, params['norm'][layer_idx][k]['g'], params['norm'][layer_idx][k]['b']), approximate=False) for k, v in out.items()}
        layer_idx += 1
    outs = [x[ntp] @ params['out'][ntp]['W'] + params['out'][ntp]['b'] for ntp in NODE_TYPES]
    return jnp.concatenate(outs, axis=0)


# speedup vs baseline: 15.9580x; 15.9580x over previous
import functools

import jax
import jax.numpy as jnp
from jax import lax
from jax.experimental import pallas as pl
from jax.experimental.pallas import tpu as pltpu
from jax.experimental.pallas import tpu_sc as plsc

NODE_TYPES = ['flow', 'time', 'attr']
ETS = [('flow', 'acts_in', 'time', 2), ('time', 'evolves_to', 'time', 2),
       ('flow', 'interacts', 'flow', 5), ('flow', 'uses', 'attr', 1)]
H = 4
C = 128
E = 131072

NC = 2
NS = 16
NW = NC * NS
CHUNK = 128
IDX_ROWS = E // CHUNK
ROWS_PER_TILE = IDX_ROWS // NW

_f32 = jnp.float32


def _pad128(n):
    return ((n + 127) // 128) * 128


def _acc_chunks(rpt):
    out = []
    off = 0
    while off < rpt:
        size = min(128, rpt - off)
        out.append((off, size))
        off += size
    return out


def _mesh():
    return plsc.VectorSubcoreMesh(core_axis_name="c", subcore_axis_name="s")


def _zero_fill(zb_v, nrows):
    def fill(i, carry):
        z16 = jnp.zeros((16,), _f32)
        for j in range(8):
            zb_v[i, pl.ds(j * 16, 16)] = z16
        return carry
    lax.fori_loop(0, nrows, fill, 0)


@functools.lru_cache(maxsize=None)
def _make_sage_agg(n_src, n_pad):
    rpt = n_pad // NS

    def body(x_hbm, src_hbm, dst_hbm, sum_out, src_v, dst_v, rows_v, zb_v, acc_sh, sem):
        c = lax.axis_index("c")
        s = lax.axis_index("s")
        wid = s * NC + c

        _zero_fill(zb_v, 128)
        base = s * rpt
        for (off, size) in _acc_chunks(rpt):
            pltpu.sync_copy(zb_v.at[pl.ds(0, size)], acc_sh.at[pl.ds(base + off, size)])
        plsc.subcore_barrier()

        t0 = wid * ROWS_PER_TILE
        pltpu.sync_copy(src_hbm.at[pl.ds(t0, ROWS_PER_TILE)], src_v)
        pltpu.sync_copy(dst_hbm.at[pl.ds(t0, ROWS_PER_TILE)], dst_v)

        def step(j, carry):
            pltpu.async_copy(x_hbm.at[src_v.at[j]], rows_v, sem).wait()
            pltpu.sync_copy(rows_v, acc_sh.at[dst_v.at[j]], add=True)
            return carry
        lax.fori_loop(0, ROWS_PER_TILE, step, 0)
        plsc.subcore_barrier()

        for (off, size) in _acc_chunks(rpt):
            pltpu.sync_copy(acc_sh.at[pl.ds(base + off, size)], rows_v.at[pl.ds(0, size)])
            pltpu.sync_copy(rows_v.at[pl.ds(0, size)], sum_out.at[c, pl.ds(base + off, size)])

    return pl.kernel(
        body,
        out_type=jax.ShapeDtypeStruct((NC, n_pad, C), _f32),
        mesh=_mesh(),
        scratch_types=[
            pltpu.VMEM((ROWS_PER_TILE, CHUNK), jnp.int32),
            pltpu.VMEM((ROWS_PER_TILE, CHUNK), jnp.int32),
            pltpu.VMEM((CHUNK, C), _f32),
            pltpu.VMEM((CHUNK, C), _f32),
            pltpu.VMEM_SHARED((n_pad, C), _f32),
            pltpu.SemaphoreType.DMA,
        ],
    )


@functools.lru_cache(maxsize=None)
def _make_count(n_pad):
    rpt = n_pad // NS

    def body(dst_hbm, cnt_out, dst_v, ones_v, zb_v, acc_sh):
        c = lax.axis_index("c")
        s = lax.axis_index("s")
        wid = s * NC + c

        _zero_fill(zb_v, 128)

        def fill1(i, carry):
            o16 = jnp.ones((16,), _f32)
            for j in range(8):
                ones_v[i, pl.ds(j * 16, 16)] = o16
            return carry
        lax.fori_loop(0, CHUNK, fill1, 0)

        base = s * rpt
        for (off, size) in _acc_chunks(rpt):
            pltpu.sync_copy(zb_v.at[pl.ds(0, size)], acc_sh.at[pl.ds(base + off, size)])
        plsc.subcore_barrier()

        t0 = wid * ROWS_PER_TILE
        pltpu.sync_copy(dst_hbm.at[pl.ds(t0, ROWS_PER_TILE)], dst_v)

        def step(j, carry):
            pltpu.sync_copy(ones_v, acc_sh.at[dst_v.at[j]], add=True)
            return carry
        lax.fori_loop(0, ROWS_PER_TILE, step, 0)
        plsc.subcore_barrier()

        for (off, size) in _acc_chunks(rpt):
            pltpu.sync_copy(acc_sh.at[pl.ds(base + off, size)], zb_v.at[pl.ds(0, size)])
            pltpu.sync_copy(zb_v.at[pl.ds(0, size)], cnt_out.at[c, pl.ds(base + off, size)])

    return pl.kernel(
        body,
        out_type=jax.ShapeDtypeStruct((NC, n_pad, C), _f32),
        mesh=_mesh(),
        scratch_types=[
            pltpu.VMEM((ROWS_PER_TILE, CHUNK), jnp.int32),
            pltpu.VMEM((CHUNK, C), _f32),
            pltpu.VMEM((CHUNK, C), _f32),
            pltpu.VMEM_SHARED((n_pad, C), _f32),
        ],
    )


@functools.lru_cache(maxsize=None)
def _make_gather(n_tab):
    def body(tab_hbm, idx_hbm, out_hbm, idx_v, rows_v, sem):
        c = lax.axis_index("c")
        s = lax.axis_index("s")
        wid = s * NC + c
        t0 = wid * ROWS_PER_TILE
        pltpu.sync_copy(idx_hbm.at[pl.ds(t0, ROWS_PER_TILE)], idx_v)

        def step(j, carry):
            pltpu.async_copy(tab_hbm.at[idx_v.at[j]], rows_v, sem).wait()
            pltpu.sync_copy(rows_v, out_hbm.at[pl.ds((t0 + j) * CHUNK, CHUNK)])
            return carry
        lax.fori_loop(0, ROWS_PER_TILE, step, 0)

    return pl.kernel(
        body,
        out_type=jax.ShapeDtypeStruct((E, C), _f32),
        mesh=_mesh(),
        scratch_types=[
            pltpu.VMEM((ROWS_PER_TILE, CHUNK), jnp.int32),
            pltpu.VMEM((CHUNK, C), _f32),
            pltpu.SemaphoreType.DMA,
        ],
    )


@functools.lru_cache(maxsize=None)
def _make_gather_heads(n_tab):
    def body(tab_hbm, idx_hbm, out_hbm, idx_v, rows_v, sem):
        c = lax.axis_index("c")
        s = lax.axis_index("s")
        wid = s * NC + c
        t0 = wid * ROWS_PER_TILE
        pltpu.sync_copy(idx_hbm.at[pl.ds(t0, ROWS_PER_TILE)], idx_v)

        def step(j, carry):
            for h in range(H):
                pltpu.async_copy(tab_hbm.at[h].at[idx_v.at[j]], rows_v, sem).wait()
                pltpu.sync_copy(rows_v, out_hbm.at[h, pl.ds((t0 + j) * CHUNK, CHUNK)])
            return carry
        lax.fori_loop(0, ROWS_PER_TILE, step, 0)

    return pl.kernel(
        body,
        out_type=jax.ShapeDtypeStruct((H, E, C), _f32),
        mesh=_mesh(),
        scratch_types=[
            pltpu.VMEM((ROWS_PER_TILE, CHUNK), jnp.int32),
            pltpu.VMEM((CHUNK, C), _f32),
            pltpu.SemaphoreType.DMA,
        ],
    )


@functools.lru_cache(maxsize=None)
def _make_scatter_add(n_pad):
    rpt = n_pad // NS

    def body(vals_hbm, idx_hbm, out_hbm, idx_v, vals_v, zb_v, acc_sh):
        c = lax.axis_index("c")
        s = lax.axis_index("s")
        wid = s * NC + c

        _zero_fill(zb_v, 128)
        base = s * rpt
        for (off, size) in _acc_chunks(rpt):
            pltpu.sync_copy(zb_v.at[pl.ds(0, size)], acc_sh.at[pl.ds(base + off, size)])
        plsc.subcore_barrier()

        t0 = wid * ROWS_PER_TILE
        pltpu.sync_copy(idx_hbm.at[pl.ds(t0, ROWS_PER_TILE)], idx_v)

        def step(j, carry):
            pltpu.sync_copy(vals_hbm.at[pl.ds((t0 + j) * CHUNK, CHUNK)], vals_v)
            pltpu.sync_copy(vals_v, acc_sh.at[idx_v.at[j]], add=True)
            return carry
        lax.fori_loop(0, ROWS_PER_TILE, step, 0)
        plsc.subcore_barrier()

        for (off, size) in _acc_chunks(rpt):
            pltpu.sync_copy(acc_sh.at[pl.ds(base + off, size)], vals_v.at[pl.ds(0, size)])
            pltpu.sync_copy(vals_v.at[pl.ds(0, size)], out_hbm.at[c, pl.ds(base + off, size)])

    return pl.kernel(
        body,
        out_type=jax.ShapeDtypeStruct((NC, n_pad, C), _f32),
        mesh=_mesh(),
        scratch_types=[
            pltpu.VMEM((ROWS_PER_TILE, CHUNK), jnp.int32),
            pltpu.VMEM((CHUNK, C), _f32),
            pltpu.VMEM((CHUNK, C), _f32),
            pltpu.VMEM_SHARED((n_pad, C), _f32),
        ],
    )


def _row_block(n):
    for r in (512, 400, 1024, 250, 200, 128, 100):
        if n % r == 0:
            return r
    return n


@functools.lru_cache(maxsize=None)
def _make_linear(n, k1, k2):
    r = _row_block(n)

    def body(x_ref, w_ref, b_ref, o_ref):
        o_ref[...] = (jnp.dot(x_ref[...], w_ref[...], preferred_element_type=_f32)
                      + b_ref[...])

    return pl.pallas_call(
        body,
        grid=(n // r,),
        in_specs=[pl.BlockSpec((r, k1), lambda i: (i, 0)),
                  pl.BlockSpec((k1, k2), lambda i: (0, 0)),
                  pl.BlockSpec((1, k2), lambda i: (0, 0))],
        out_specs=pl.BlockSpec((r, k2), lambda i: (i, 0)),
        out_shape=jax.ShapeDtypeStruct((n, k2), _f32),
    )


def _linear(x, w, b=None):
    n, k1 = x.shape
    k2 = w.shape[1]
    b2 = jnp.zeros((1, k2), _f32) if b is None else b.reshape(1, k2)
    return _make_linear(n, k1, k2)(x, w, b2)


@functools.lru_cache(maxsize=None)
def _make_linear_heads(n):
    r = _row_block(n)

    def body(x_ref, w_ref, o_ref):
        o_ref[...] = jnp.dot(x_ref[...], w_ref[...],
                             preferred_element_type=_f32)[None]

    return pl.pallas_call(
        body,
        grid=(H, n // r),
        in_specs=[pl.BlockSpec((r, C), lambda h, i: (i, 0)),
                  pl.BlockSpec((C, C), lambda h, i: (0, h))],
        out_specs=pl.BlockSpec((1, r, C), lambda h, i: (h, i, 0)),
        out_shape=jax.ShapeDtypeStruct((H, n, C), _f32),
    )


@functools.lru_cache(maxsize=None)
def _make_sage_post(n, n_pad):
    r = _row_block(n)

    def body(sp_ref, cp_ref, xd_ref, wl_ref, bl_ref, wr_ref, o_ref):
        sp = sp_ref[...]
        cnt = cp_ref[0, :, 0:1] + cp_ref[1, :, 0:1]
        mean = (sp[0] + sp[1]) / jnp.clip(cnt, 1.0, None)
        o_ref[...] = (jnp.dot(mean, wl_ref[...], preferred_element_type=_f32)
                      + bl_ref[...]
                      + jnp.dot(xd_ref[...], wr_ref[...], preferred_element_type=_f32))

    return pl.pallas_call(
        body,
        grid=(n // r,),
        in_specs=[pl.BlockSpec((NC, r, C), lambda i: (0, i, 0)),
                  pl.BlockSpec((NC, r, C), lambda i: (0, i, 0)),
                  pl.BlockSpec((r, C), lambda i: (i, 0)),
                  pl.BlockSpec((C, C), lambda i: (0, 0)),
                  pl.BlockSpec((1, C), lambda i: (0, 0)),
                  pl.BlockSpec((C, C), lambda i: (0, 0))],
        out_specs=pl.BlockSpec((r, C), lambda i: (i, 0)),
        out_shape=jax.ShapeDtypeStruct((n, C), _f32),
    )


@functools.lru_cache(maxsize=None)
def _make_ln_gelu(n):
    r = _row_block(n)

    def body(x_ref, v_ref, g_ref, b_ref, o_ref):
        y = x_ref[...] + v_ref[...]
        mu = jnp.mean(y, axis=1, keepdims=True)
        yc = y - mu
        var = jnp.mean(yc * yc, axis=1, keepdims=True)
        t = yc * lax.rsqrt(var + 1e-5) * g_ref[...] + b_ref[...]
        o_ref[...] = 0.5 * t * (1.0 + lax.erf(t * 0.7071067811865476))

    return pl.pallas_call(
        body,
        grid=(n // r,),
        in_specs=[pl.BlockSpec((r, C), lambda i: (i, 0)),
                  pl.BlockSpec((r, C), lambda i: (i, 0)),
                  pl.BlockSpec((1, C), lambda i: (0, 0)),
                  pl.BlockSpec((1, C), lambda i: (0, 0))],
        out_specs=pl.BlockSpec((r, C), lambda i: (i, 0)),
        out_shape=jax.ShapeDtypeStruct((n, C), _f32),
    )


ER = 1024


@functools.lru_cache(maxsize=None)
def _make_alpha(ed):
    def body(xls_ref, xrd_ref, ea_ref, we_ref, att_ref, o_ref):
        ea = ea_ref[...]
        lane = lax.broadcasted_iota(jnp.int32, (ER, C), 1)
        out = jnp.zeros((ER, C), _f32)
        for h in range(H):
            z = xls_ref[h] + xrd_ref[h]
            for k in range(ed):
                z = z + ea[:, k:k + 1] * we_ref[k, h:h + 1, :]
            z = jnp.where(z >= 0, z, 0.2 * z)
            ah = jnp.sum(z * att_ref[h:h + 1, :], axis=1, keepdims=True)
            out = jnp.where(lane == h, jnp.exp(ah), out)
        o_ref[...] = out

    return pl.pallas_call(
        body,
        grid=(E // ER,),
        in_specs=[pl.BlockSpec((H, ER, C), lambda i: (0, i, 0)),
                  pl.BlockSpec((H, ER, C), lambda i: (0, i, 0)),
                  pl.BlockSpec((ER, ed), lambda i: (i, 0)),
                  pl.BlockSpec((ed, H, C), lambda i: (0, 0, 0)),
                  pl.BlockSpec((H, C), lambda i: (0, 0))],
        out_specs=pl.BlockSpec((ER, C), lambda i: (i, 0)),
        out_shape=jax.ShapeDtypeStruct((E, C), _f32),
    )


@functools.lru_cache(maxsize=None)
def _make_combine():
    def body(xls_ref, p_ref, sd_ref, o_ref):
        pe = p_ref[...]
        sd = sd_ref[...]
        out = jnp.zeros((ER, C), _f32)
        for h in range(H):
            coef = pe[:, h:h + 1] / (sd[:, h:h + 1] + 1e-16)
            out = out + coef * xls_ref[h]
        o_ref[...] = out

    return pl.pallas_call(
        body,
        grid=(E // ER,),
        in_specs=[pl.BlockSpec((H, ER, C), lambda i: (0, i, 0)),
                  pl.BlockSpec((ER, C), lambda i: (i, 0)),
                  pl.BlockSpec((ER, C), lambda i: (i, 0))],
        out_specs=pl.BlockSpec((ER, C), lambda i: (i, 0)),
        out_shape=jax.ShapeDtypeStruct((E, C), _f32),
    )


def _sage_layer(x, eidx, cnts, params_l):
    out = {}
    for (s, name, d, ed) in ETS:
        srcR, dstR = eidx[name]
        p = params_l[name]
        n_dst = x[d].shape[0]
        sum_p = _make_sage_agg(x[s].shape[0], _pad128(n_dst))(x[s], srcR, dstR)
        o = _make_sage_post(n_dst, _pad128(n_dst))(sum_p, cnts[name], x[d], p['Wl'],
                                                   p['bl'].reshape(1, C), p['Wr'])
        out[d] = out.get(d, 0.0) + o
    return out


def _gat_layer(x, eidx, eas, params_l):
    out = {}
    for (s, name, d, ed) in ETS:
        srcR, dstR = eidx[name]
        p = params_l[name]
        n_src, n_dst = x[s].shape[0], x[d].shape[0]
        n_pad = _pad128(n_dst)
        xl = _make_linear_heads(n_src)(x[s], p['Wl'])
        xr = _make_linear_heads(n_dst)(x[d], p['Wr'])
        xls = _make_gather_heads(n_src)(xl, srcR)
        xrd = _make_gather_heads(n_dst)(xr, dstR)
        pe = _make_alpha(ed)(xls, xrd, eas[name], p['We'].reshape(ed, H, C), p['att'])
        sp = _make_scatter_add(n_pad)(pe, dstR)
        ssum = sp[0] + sp[1]
        sd = _make_gather(n_pad)(ssum, dstR)
        v = _make_combine()(xls, pe, sd)
        vp = _make_scatter_add(n_pad)(v, dstR)
        o = (vp[0, :n_dst] + vp[1, :n_dst]) * (1.0 / H) + p['bias']
        out[d] = out.get(d, 0.0) + o
    return out


def kernel(x_flow, x_time, x_attr, ei_acts_in, ei_evolves_to, ei_interacts,
           ei_uses, ea_acts_in, ea_evolves_to, ea_interacts, ea_uses, params):
    eis = {'acts_in': ei_acts_in, 'evolves_to': ei_evolves_to,
           'interacts': ei_interacts, 'uses': ei_uses}
    eas = {'acts_in': ea_acts_in, 'evolves_to': ea_evolves_to,
           'interacts': ea_interacts, 'uses': ea_uses}
    eidx = {name: (eis[name][0].reshape(IDX_ROWS, CHUNK),
                   eis[name][1].reshape(IDX_ROWS, CHUNK)) for (_, name, _, _) in ETS}
    n_of = {'flow': x_flow.shape[0], 'time': x_time.shape[0], 'attr': x_attr.shape[0]}
    cnts = {name: _make_count(_pad128(n_of[d]))(eidx[name][1])
            for (_, name, d, _) in ETS}

    x = {'flow': _linear(x_flow, params['in']['flow']['W'], params['in']['flow']['b']),
         'time': _linear(x_time, params['in']['time']['W'], params['in']['time']['b']),
         'attr': _linear(x_attr, params['in']['attr']['W'], params['in']['attr']['b'])}

    layer_idx = 0
    for li in range(2):
        out = _sage_layer(x, eidx, cnts, params['sage'][li])
        x = {k: _make_ln_gelu(x[k].shape[0])(
                 x[k], v, params['norm'][layer_idx][k]['g'].reshape(1, C),
                 params['norm'][layer_idx][k]['b'].reshape(1, C))
             for k, v in out.items()}
        layer_idx += 1
    for li in range(2):
        out = _gat_layer(x, eidx, eas, params['gat'][li])
        x = {k: _make_ln_gelu(x[k].shape[0])(
                 x[k], v, params['norm'][layer_idx][k]['g'].reshape(1, C),
                 params['norm'][layer_idx][k]['b'].reshape(1, C))
             for k, v in out.items()}
        layer_idx += 1

    outs = [_linear(x[ntp], params['out'][ntp]['W'], params['out'][ntp]['b'])
            for ntp in NODE_TYPES]
    return jnp.concatenate(outs, axis=0)
